# baseline (speedup 1.0000x reference)
import functools

import jax
import jax.numpy as jnp
from jax import lax
from jax.experimental import pallas as pl
from jax.experimental.pallas import tpu as pltpu
from jax.experimental.pallas import tpu_sc as plsc

E = 64
H = 768
HD = 384
CAP = 160
TBLK = 1024
DUMP = E * CAP
BUF_ROWS = E * CAP + 8
EBLK = 4
Y_ROWS = (E + EBLK) * CAP

NW = 32
CHUNK = 64
NCH = 4


SUB = 256
NSUB = TBLK // SUB


def _routing_kernel(x_ref, gw_ref, slot_ref, carry_ref):
    i = pl.program_id(0)

    @pl.when(i == 0)
    def _():
        carry_ref[...] = jnp.zeros_like(carry_ref)

    carry = carry_ref[...]
    jj = lax.broadcasted_iota(jnp.int32, (SUB, SUB), 0)
    ii = lax.broadcasted_iota(jnp.int32, (SUB, SUB), 1)
    upper = (jj < ii).astype(jnp.bfloat16)
    for j in range(NSUB):
        xs = x_ref[j * SUB:(j + 1) * SUB, :]
        logits = lax.dot_general(gw_ref[...], xs, (((1,), (1,)), ((), ())),
                                 preferred_element_type=jnp.float32)
        m = jnp.max(logits, axis=0, keepdims=True)
        eio = lax.broadcasted_iota(jnp.int32, (E, SUB), 0)
        e_idx = jnp.min(jnp.where(logits == m, eio, E), axis=0, keepdims=True)
        onehot = eio == e_idx
        prior = lax.dot_general(onehot.astype(jnp.bfloat16), upper,
                                (((1,), (0,)), ((), ())),
                                preferred_element_type=jnp.float32)
        oh_f = onehot.astype(jnp.float32)
        prior_own = jnp.sum(prior * oh_f, axis=0, keepdims=True)
        base_own = jnp.sum(carry * oh_f, axis=0, keepdims=True)
        pos = base_own + prior_own
        e_f = e_idx.astype(jnp.float32)
        slot = jnp.where(pos < float(CAP), e_f * float(CAP) + pos, float(DUMP))
        slot_ref[0, 0, j * SUB:(j + 1) * SUB] = slot.astype(jnp.int32).reshape(SUB)
        carry = carry + jnp.sum(oh_f, axis=1, keepdims=True)
    carry_ref[...] = carry


def _routing(x_flat, gate_w, interpret=False):
    nblk = x_flat.shape[0] // TBLK
    return pl.pallas_call(
        _routing_kernel,
        grid=(nblk,),
        in_specs=[
            pl.BlockSpec((TBLK, H), lambda i: (i, 0)),
            pl.BlockSpec((E, H), lambda i: (0, 0)),
        ],
        out_specs=pl.BlockSpec((1, 1, TBLK), lambda i: (i, 0, 0)),
        out_shape=jax.ShapeDtypeStruct((nblk, 1, TBLK), jnp.int32),
        scratch_shapes=[pltpu.VMEM((E, 1), jnp.float32)],
        interpret=interpret,
    )(x_flat, gate_w)


_SC_SCRATCH = (
    [pltpu.VMEM((CHUNK,), jnp.int32) for _ in range(NCH)]
    + [pltpu.VMEM((CHUNK, H), jnp.float32) for _ in range(2)]
    + [pltpu.SemaphoreType.DMA for _ in range(4)]
)


def _dispatch(x_flat, slot):
    t = x_flat.shape[0]
    tpw = t // NW
    assert tpw == NCH * CHUNK

    @functools.partial(
        pl.kernel,
        out_type=jax.ShapeDtypeStruct((BUF_ROWS, H), jnp.float32),
        mesh=plsc.VectorSubcoreMesh(core_axis_name="c", subcore_axis_name="s"),
        scratch_types=list(_SC_SCRATCH),
    )
    def run(x_hbm, slot_hbm, buf_hbm, i0, i1, i2, i3, rb0, rb1,
            ls0, ls1, ss0, ss1):
        wid = lax.axis_index("s") * 2 + lax.axis_index("c")
        base = wid * tpw
        idx = [i0, i1, i2, i3]
        rb, ls, ss = [rb0, rb1], [ls0, ls1], [ss0, ss1]
        for c in range(NCH):
            pltpu.sync_copy(slot_hbm.at[pl.ds(base + c * CHUNK, CHUNK)], idx[c])
        loads = [None] * NCH
        stores = [None] * NCH
        for c in range(2):
            loads[c] = pltpu.async_copy(
                x_hbm.at[pl.ds(base + c * CHUNK, CHUNK)], rb[c], ls[c])
        for c in range(NCH):
            b = c % 2
            loads[c].wait()
            stores[c] = pltpu.async_copy(rb[b], buf_hbm.at[idx[c]], ss[b])
            if c + 2 < NCH:
                stores[c].wait()
                loads[c + 2] = pltpu.async_copy(
                    x_hbm.at[pl.ds(base + (c + 2) * CHUNK, CHUNK)], rb[b], ls[b])
        stores[NCH - 2].wait()
        stores[NCH - 1].wait()

    return run(x_flat, slot)


def _expert_mm_kernel(a_ref, w1_ref, w2_ref, y_ref):
    g = pl.program_id(0)
    ng = E // EBLK

    @pl.when(g < ng)
    def _():
        for k in range(EBLK):
            a = a_ref[k * CAP:(k + 1) * CAP, :]
            y_ref[k * CAP:(k + 1) * CAP, :HD] = lax.dot_general(
                a, w1_ref[k], (((1,), (1,)), ((), ())),
                preferred_element_type=jnp.float32)
            y_ref[k * CAP:(k + 1) * CAP, HD:] = lax.dot_general(
                a, w2_ref[k], (((1,), (1,)), ((), ())),
                preferred_element_type=jnp.float32)

    @pl.when(g == ng)
    def _():
        y_ref[...] = jnp.zeros_like(y_ref)


def _expert_mm(buf, expert_w, interpret=False):
    ng = E // EBLK
    return pl.pallas_call(
        _expert_mm_kernel,
        grid=(ng + 1,),
        in_specs=[
            pl.BlockSpec((EBLK * CAP, H), lambda g: (jnp.minimum(g, ng - 1), 0)),
            pl.BlockSpec((EBLK, HD, H), lambda g: (jnp.minimum(g, ng - 1), 0, 0)),
            pl.BlockSpec((EBLK, HD, H), lambda g: (jnp.minimum(g, ng - 1), 1, 0)),
        ],
        out_specs=pl.BlockSpec((EBLK * CAP, H), lambda g: (g, 0)),
        out_shape=jax.ShapeDtypeStruct((Y_ROWS, H), jnp.float32),
        interpret=interpret,
    )(buf, expert_w, expert_w)


def _combine(y, slot):
    t = slot.shape[0]
    tpw = t // NW

    assert tpw == NCH * CHUNK

    @functools.partial(
        pl.kernel,
        out_type=jax.ShapeDtypeStruct((t, H), jnp.float32),
        mesh=plsc.VectorSubcoreMesh(core_axis_name="c", subcore_axis_name="s"),
        scratch_types=list(_SC_SCRATCH),
    )
    def run(y_hbm, slot_hbm, out_hbm, i0, i1, i2, i3, rb0, rb1,
            ls0, ls1, ss0, ss1):
        wid = lax.axis_index("s") * 2 + lax.axis_index("c")
        base = wid * tpw
        idx = [i0, i1, i2, i3]
        rb, ls, ss = [rb0, rb1], [ls0, ls1], [ss0, ss1]
        for c in range(NCH):
            pltpu.sync_copy(slot_hbm.at[pl.ds(base + c * CHUNK, CHUNK)], idx[c])
        loads = [None] * NCH
        stores = [None] * NCH
        for c in range(2):
            loads[c] = pltpu.async_copy(y_hbm.at[idx[c]], rb[c], ls[c])
        for c in range(NCH):
            b = c % 2
            loads[c].wait()
            stores[c] = pltpu.async_copy(
                rb[b], out_hbm.at[pl.ds(base + c * CHUNK, CHUNK)], ss[b])
            if c + 2 < NCH:
                stores[c].wait()
                loads[c + 2] = pltpu.async_copy(y_hbm.at[idx[c + 2]], rb[b], ls[b])
        stores[NCH - 2].wait()
        stores[NCH - 1].wait()

    return run(y, slot)


def kernel(x, gate_w, expert_w):
    b, s, h = x.shape
    x_flat = x.reshape(b * s, h)
    slot = _routing(x_flat, gate_w).reshape(b * s)
    buf = _dispatch(x_flat, slot)
    y = _expert_mm(buf, expert_w)
    out = _combine(y, slot)
    return out.reshape(b, s, h)


# speedup vs baseline: 1.0440x; 1.0440x over previous
import functools

import jax
import jax.numpy as jnp
from jax import lax
from jax.experimental import pallas as pl
from jax.experimental.pallas import tpu as pltpu
from jax.experimental.pallas import tpu_sc as plsc

E = 64
H = 768
HP = 384
HD = 384
CAP = 160
TBLK = 1024
DUMP = E * CAP
BUF_ROWS = E * CAP + 8
EBLK = 4
Y_ROWS = (E + EBLK) * CAP

NW = 32
CHUNK = 128
NCH = 2


SUB = 256
NSUB = TBLK // SUB


def _routing_kernel(x_ref, gw_ref, slot_ref, xbf_ref, carry_ref):
    i = pl.program_id(0)

    @pl.when(i == 0)
    def _():
        carry_ref[...] = jnp.zeros_like(carry_ref)

    xi = lax.bitcast_convert_type(x_ref[...], jnp.int32)
    rnd = lambda r: ((r + 0x7FFF + ((r >> 16) & 1)) >> 16) & 0xFFFF
    xbf_ref[...] = rnd(xi[:, :HP]) | (rnd(xi[:, HP:]) << 16)
    carry = carry_ref[...]
    jj = lax.broadcasted_iota(jnp.int32, (SUB, SUB), 0)
    ii = lax.broadcasted_iota(jnp.int32, (SUB, SUB), 1)
    upper = (jj < ii).astype(jnp.bfloat16)
    for j in range(NSUB):
        xs = x_ref[j * SUB:(j + 1) * SUB, :]
        logits = lax.dot_general(gw_ref[...], xs, (((1,), (1,)), ((), ())),
                                 preferred_element_type=jnp.float32)
        m = jnp.max(logits, axis=0, keepdims=True)
        eio = lax.broadcasted_iota(jnp.int32, (E, SUB), 0)
        e_idx = jnp.min(jnp.where(logits == m, eio, E), axis=0, keepdims=True)
        onehot = eio == e_idx
        prior = lax.dot_general(onehot.astype(jnp.bfloat16), upper,
                                (((1,), (0,)), ((), ())),
                                preferred_element_type=jnp.float32)
        oh_f = onehot.astype(jnp.float32)
        prior_own = jnp.sum(prior * oh_f, axis=0, keepdims=True)
        base_own = jnp.sum(carry * oh_f, axis=0, keepdims=True)
        pos = base_own + prior_own
        e_f = e_idx.astype(jnp.float32)
        slot = jnp.where(pos < float(CAP), e_f * float(CAP) + pos, float(DUMP))
        slot_ref[0, 0, j * SUB:(j + 1) * SUB] = slot.astype(jnp.int32).reshape(SUB)
        carry = carry + jnp.sum(oh_f, axis=1, keepdims=True)
    carry_ref[...] = carry


def _routing(x_flat, gate_w, interpret=False):
    nblk = x_flat.shape[0] // TBLK
    return pl.pallas_call(
        _routing_kernel,
        grid=(nblk,),
        in_specs=[
            pl.BlockSpec((TBLK, H), lambda i: (i, 0)),
            pl.BlockSpec((E, H), lambda i: (0, 0)),
        ],
        out_specs=[
            pl.BlockSpec((1, 1, TBLK), lambda i: (i, 0, 0)),
            pl.BlockSpec((TBLK, HP), lambda i: (i, 0)),
        ],
        out_shape=[
            jax.ShapeDtypeStruct((nblk, 1, TBLK), jnp.int32),
            jax.ShapeDtypeStruct((nblk * TBLK, HP), jnp.int32),
        ],
        scratch_shapes=[pltpu.VMEM((E, 1), jnp.float32)],
        interpret=interpret,
    )(x_flat, gate_w)


def _dispatch(x_pk, slot):
    t = x_pk.shape[0]
    tpw = t // NW

    @functools.partial(
        pl.kernel,
        out_type=jax.ShapeDtypeStruct((BUF_ROWS, HP), jnp.int32),
        mesh=plsc.VectorSubcoreMesh(core_axis_name="c", subcore_axis_name="s"),
        scratch_types=[
            pltpu.VMEM((tpw,), jnp.int32),
            pltpu.VMEM((tpw, HP), jnp.int32),
            pltpu.SemaphoreType.DMA,
        ],
    )
    def run(x_hbm, slot_hbm, buf_hbm, idx_v, rows_v, sem):
        wid = lax.axis_index("s") * 2 + lax.axis_index("c")
        base = wid * tpw
        pltpu.sync_copy(slot_hbm.at[pl.ds(base, tpw)], idx_v)
        pltpu.sync_copy(x_hbm.at[pl.ds(base, tpw)], rows_v)
        pltpu.async_copy(rows_v, buf_hbm.at[idx_v], sem).wait()

    return run(x_pk, slot)


def _expert_mm_kernel(a_ref, w1_ref, w2_ref, y_ref):
    g = pl.program_id(0)
    ng = E // EBLK

    @pl.when(g < ng)
    def _():
        for k in range(EBLK):
            ai = a_ref[k * CAP:(k + 1) * CAP, :]
            a_lo = lax.bitcast_convert_type(ai << 16, jnp.float32)
            a_hi = lax.bitcast_convert_type(ai & jnp.int32(-65536),
                                            jnp.float32)
            for half, w_ref in ((0, w1_ref), (1, w2_ref)):
                acc = lax.dot_general(
                    a_lo, w_ref[k][:, :HP], (((1,), (1,)), ((), ())),
                    preferred_element_type=jnp.float32)
                acc += lax.dot_general(
                    a_hi, w_ref[k][:, HP:], (((1,), (1,)), ((), ())),
                    preferred_element_type=jnp.float32)
                y_ref[k * CAP:(k + 1) * CAP,
                      half * HD:(half + 1) * HD] = acc

    @pl.when(g == ng)
    def _():
        y_ref[...] = jnp.zeros_like(y_ref)


def _expert_mm(buf, expert_w, interpret=False):
    ng = E // EBLK
    return pl.pallas_call(
        _expert_mm_kernel,
        grid=(ng + 1,),
        in_specs=[
            pl.BlockSpec((EBLK * CAP, HP), lambda g: (jnp.minimum(g, ng - 1), 0)),
            pl.BlockSpec((EBLK, HD, H), lambda g: (jnp.minimum(g, ng - 1), 0, 0)),
            pl.BlockSpec((EBLK, HD, H), lambda g: (jnp.minimum(g, ng - 1), 1, 0)),
        ],
        out_specs=pl.BlockSpec((EBLK * CAP, H), lambda g: (g, 0)),
        out_shape=jax.ShapeDtypeStruct((Y_ROWS, H), jnp.float32),
        interpret=interpret,
    )(buf, expert_w, expert_w)


def _combine(y, slot):
    t = slot.shape[0]
    tpw = t // NW

    assert tpw == NCH * CHUNK

    @functools.partial(
        pl.kernel,
        out_type=jax.ShapeDtypeStruct((t, H), jnp.float32),
        mesh=plsc.VectorSubcoreMesh(core_axis_name="c", subcore_axis_name="s"),
        scratch_types=[
            pltpu.VMEM((CHUNK,), jnp.int32),
            pltpu.VMEM((CHUNK, H), jnp.float32),
            pltpu.SemaphoreType.DMA,
        ],
    )
    def run(y_hbm, slot_hbm, out_hbm, idx_v, rows_v, sem):
        wid = lax.axis_index("s") * 2 + lax.axis_index("c")
        for c in range(NCH):
            base = wid * tpw + c * CHUNK
            pltpu.sync_copy(slot_hbm.at[pl.ds(base, CHUNK)], idx_v)
            pltpu.async_copy(y_hbm.at[idx_v], rows_v, sem).wait()
            pltpu.sync_copy(rows_v, out_hbm.at[pl.ds(base, CHUNK)])

    return run(y, slot)


def kernel(x, gate_w, expert_w):
    b, s, h = x.shape
    x_flat = x.reshape(b * s, h)
    slot3, x_bf = _routing(x_flat, gate_w)
    slot = slot3.reshape(b * s)
    buf = _dispatch(x_bf, slot)
    y = _expert_mm(buf, expert_w)
    out = _combine(y, slot)
    return out.reshape(b, s, h)
